# baseline (speedup 1.0000x reference)
import jax
import jax.numpy as jnp
from jax.experimental import pallas as pl
from jax.experimental.pallas import tpu as pltpu
from jax.experimental.pallas import tpu_sc as plsc

H = 128
W = 128
CF = 16
K = 8
N = H * W
BQ = 128
SQ = 32
CW = 1024
NT = N // CW


def _conv(x, w, b):
    y = jax.lax.conv_general_dilated(
        x, w, (1, 1), 'SAME', dimension_numbers=('NCHW', 'OIHW', 'NCHW'))
    return y + b[None, :, None, None]


AW = 128
NSUB = CW // AW


def _topk_body(q_ref, kt_ref, av_ref, ai_ref, cost_scr):
    q = q_ref[...]
    q2 = jnp.sum(q * q, axis=1, keepdims=True)

    inf = jnp.float32(jnp.inf)
    lane = jax.lax.broadcasted_iota(
        jnp.int32, (SQ, AW), 1).astype(jnp.float32)

    def mm_step(t, carry):
        kt = kt_ref[t]
        k2 = jnp.sum(kt * kt, axis=0, keepdims=True)
        c = q2 - 2.0 * jnp.dot(q, kt, preferred_element_type=jnp.float32) + k2
        for s in range(NSUB):
            cost_scr[t * NSUB + s] = c[:, s * AW:(s + 1) * AW]
        return carry

    jax.lax.fori_loop(0, NT, mm_step, 0, unroll=2)

    for qb in range(BQ // SQ):
        acc_v0 = tuple(jnp.full((SQ, AW), inf, jnp.float32)
                       for _ in range(K))
        acc_i0 = tuple(jnp.full((SQ, AW), 2.0 ** 30, jnp.float32)
                       for _ in range(K))

        def ins_step(u, carry):
            av, ai = carry
            av = list(av)
            ai = list(ai)
            x = cost_scr[u, qb * SQ:(qb + 1) * SQ, :]
            gx = lane + (u * AW).astype(jnp.float32)
            m = [x < av[j] for j in range(K)]
            for j in range(K - 1, 0, -1):
                av[j] = jnp.where(m[j - 1], av[j - 1],
                                  jnp.where(m[j], x, av[j]))
                ai[j] = jnp.where(m[j - 1], ai[j - 1],
                                  jnp.where(m[j], gx, ai[j]))
            av[0] = jnp.where(m[0], x, av[0])
            ai[0] = jnp.where(m[0], gx, ai[0])
            return tuple(av), tuple(ai)

        av, ai = jax.lax.fori_loop(0, NT * NSUB, ins_step,
                                   (acc_v0, acc_i0))
        av_ref[qb * SQ:(qb + 1) * SQ, :] = jnp.concatenate(av, axis=1)
        ai_ref[qb * SQ:(qb + 1) * SQ, :] = jnp.concatenate(ai, axis=1)


BQM = 256


def _merge_body(av_ref, ai_ref, wgt_ref, idx_ref):
    fv = av_ref[...]
    fi = ai_ref[...]
    inf = jnp.float32(jnp.inf)
    ms = []
    idxs = []
    m_prev = jnp.full((BQM, 1), -inf, dtype=jnp.float32)
    i_prev = jnp.full((BQM, 1), -1.0, dtype=jnp.float32)
    for k in range(K):
        valid = (fv > m_prev) | ((fv == m_prev) & (fi > i_prev))
        ceff = jnp.where(valid, fv, inf)
        m = jnp.min(ceff, axis=1, keepdims=True)
        i = jnp.min(jnp.where(ceff == m, fi, jnp.float32(2.0 ** 30)),
                    axis=1, keepdims=True)
        ms.append(m)
        idxs.append(i)
        m_prev, i_prev = m, i

    costs = jnp.concatenate(ms, axis=1)
    ids = jnp.concatenate(idxs, axis=1)
    e = jnp.exp(costs[:, 0:1] - costs)
    wgt_ref[...] = e / jnp.sum(e, axis=1, keepdims=True)
    idx_ref[...] = ids.astype(jnp.int32)


def _topk(qf, kt3):
    av, ai = pl.pallas_call(
        _topk_body,
        grid=(N // BQ,),
        in_specs=[
            pl.BlockSpec((BQ, CF), lambda i: (i, 0)),
            pl.BlockSpec((NT, CF, CW), lambda i: (0, 0, 0)),
        ],
        out_specs=[
            pl.BlockSpec((BQ, K * AW), lambda i: (i, 0)),
            pl.BlockSpec((BQ, K * AW), lambda i: (i, 0)),
        ],
        out_shape=[
            jax.ShapeDtypeStruct((N, K * AW), jnp.float32),
            jax.ShapeDtypeStruct((N, K * AW), jnp.float32),
        ],
        scratch_shapes=[pltpu.VMEM((N // AW, BQ, AW), jnp.float32)],
    )(qf, kt3)
    return pl.pallas_call(
        _merge_body,
        grid=(N // BQM,),
        in_specs=[
            pl.BlockSpec((BQM, K * AW), lambda i: (i, 0)),
            pl.BlockSpec((BQM, K * AW), lambda i: (i, 0)),
        ],
        out_specs=[
            pl.BlockSpec((BQM, K), lambda i: (i, 0)),
            pl.BlockSpec((BQM, K), lambda i: (i, 0)),
        ],
        out_shape=[
            jax.ShapeDtypeStruct((N, K), jnp.float32),
            jax.ShapeDtypeStruct((N, K), jnp.int32),
        ],
    )(av, ai)


def _sc_att(vT, idx, wgt):
    NI = N * K
    QW = 32
    VD = 128
    vpad = jnp.pad(vT, ((0, 0), (0, VD - CF)))
    wB = jnp.broadcast_to(wgt.reshape(N, K, 1), (N, K, CF)).reshape(N, K * CF)
    mesh = plsc.VectorSubcoreMesh(core_axis_name="core",
                                  subcore_axis_name="subcore")

    @pl.kernel(out_type=jax.ShapeDtypeStruct((N // 8, VD), jnp.float32),
               mesh=mesh,
               scratch_types=[pltpu.VMEM((QW * K, VD), jnp.float32)])
    def gk(v_hbm, i_hbm, w_hbm, o_hbm, g_scr):
        def body(i_vmem, w_vmem, o_vmem):
            pltpu.sync_copy(v_hbm.at[i_vmem.at[0]], g_scr)

            @pl.loop(0, QW // 8)
            def _(q0):
                for qq in range(8):
                    q = q0 * 8 + qq
                    acc = w_vmem[q, 0:CF] * g_scr[q * K, 0:CF]
                    for k in range(1, K):
                        acc = acc + (w_vmem[q, k * CF:(k + 1) * CF]
                                     * g_scr[q * K + k, 0:CF])
                    o_vmem[q0, qq * CF:(qq + 1) * CF] = acc

        pltpu.emit_pipeline(
            body,
            grid=(NI // (QW * K),),
            in_specs=[pl.BlockSpec((1, QW * K), index_map=lambda i: (0, i)),
                      pl.BlockSpec((QW, VD), index_map=lambda i: (i, 0))],
            out_specs=[pl.BlockSpec((QW // 8, VD), index_map=lambda i: (i, 0))],
            core_axis_name='subcore',
            dimension_semantics=(pltpu.PARALLEL,),
        )(i_hbm, w_hbm, o_hbm)

    out = gk(vpad, idx.reshape(1, NI), wB)
    return out.reshape(N, CF)


def kernel(a, b, Wq, bq, Wk, bk, Wv, bv, Wf, bf):
    q = jax.nn.relu(_conv(a, Wq, bq))[0]
    k = jax.nn.relu(_conv(b, Wk, bk))[0]
    v = jax.nn.relu(_conv(b, Wv, bv))[0]

    qf = q.reshape(CF, N).T
    kt3 = k.reshape(CF, NT, CW).transpose(1, 0, 2)
    vT = v.reshape(CF, N).T

    wgt, idx = _topk(qf, kt3)

    att = _sc_att(vT, idx, wgt)
    att = att.T.reshape(1, CF, H, W)

    out = jax.nn.sigmoid(_conv(jnp.concatenate([a, att], axis=1), Wf, bf))
    return out


# speedup vs baseline: 1.3682x; 1.3682x over previous
import jax
import jax.numpy as jnp
from jax.experimental import pallas as pl
from jax.experimental.pallas import tpu as pltpu
from jax.experimental.pallas import tpu_sc as plsc

H = 128
W = 128
CF = 16
K = 8
N = H * W
BQ = 128
SQ = 16
CW = 1024
NT = N // CW


def _conv(x, w, b):
    y = jax.lax.conv_general_dilated(
        x, w, (1, 1), 'SAME', dimension_numbers=('NCHW', 'OIHW', 'NCHW'))
    return y + b[None, :, None, None]


AW = 128
NSUB = CW // AW


def _topk_body(q_ref, kt_ref, av_ref, ai_ref, cost_scr):
    q = q_ref[...]
    q2 = jnp.sum(q * q, axis=1, keepdims=True)

    inf = jnp.float32(jnp.inf)
    lane = jax.lax.broadcasted_iota(
        jnp.int32, (SQ, AW), 1).astype(jnp.float32)

    def mm_step(t, carry):
        kt = kt_ref[t]
        k2 = jnp.sum(kt * kt, axis=0, keepdims=True)
        c = q2 - 2.0 * jnp.dot(q, kt, preferred_element_type=jnp.float32) + k2
        for s in range(NSUB):
            cost_scr[t * NSUB + s] = c[:, s * AW:(s + 1) * AW]
        return carry

    jax.lax.fori_loop(0, NT, mm_step, 0, unroll=2)

    for qb in range(BQ // SQ):
        acc_v0 = tuple(jnp.full((SQ, AW), inf, jnp.float32)
                       for _ in range(K))
        acc_i0 = tuple(jnp.full((SQ, AW), 2.0 ** 30, jnp.float32)
                       for _ in range(K))

        def ins_step(u, carry):
            av, ai = carry
            av = list(av)
            ai = list(ai)
            x = cost_scr[u, qb * SQ:(qb + 1) * SQ, :]
            gx = lane + (u * AW).astype(jnp.float32)
            m = [x < av[j] for j in range(K)]
            for j in range(K - 1, 0, -1):
                av[j] = jnp.where(m[j - 1], av[j - 1],
                                  jnp.where(m[j], x, av[j]))
                ai[j] = jnp.where(m[j - 1], ai[j - 1],
                                  jnp.where(m[j], gx, ai[j]))
            av[0] = jnp.where(m[0], x, av[0])
            ai[0] = jnp.where(m[0], gx, ai[0])
            return tuple(av), tuple(ai)

        av, ai = jax.lax.fori_loop(0, NT * NSUB, ins_step,
                                   (acc_v0, acc_i0), unroll=2)
        av_ref[qb * SQ:(qb + 1) * SQ, :] = jnp.concatenate(av, axis=1)
        ai_ref[qb * SQ:(qb + 1) * SQ, :] = jnp.concatenate(ai, axis=1)


BQM = 256


def _merge_body(av_ref, ai_ref, wgt_ref, idx_ref):
    fv = av_ref[...]
    fi = ai_ref[...]
    inf = jnp.float32(jnp.inf)
    ms = []
    idxs = []
    m_prev = jnp.full((BQM, 1), -inf, dtype=jnp.float32)
    i_prev = jnp.full((BQM, 1), -1.0, dtype=jnp.float32)
    for k in range(K):
        valid = (fv > m_prev) | ((fv == m_prev) & (fi > i_prev))
        ceff = jnp.where(valid, fv, inf)
        m = jnp.min(ceff, axis=1, keepdims=True)
        i = jnp.min(jnp.where(ceff == m, fi, jnp.float32(2.0 ** 30)),
                    axis=1, keepdims=True)
        ms.append(m)
        idxs.append(i)
        m_prev, i_prev = m, i

    costs = jnp.concatenate(ms, axis=1)
    ids = jnp.concatenate(idxs, axis=1)
    e = jnp.exp(costs[:, 0:1] - costs)
    wgt_ref[...] = e / jnp.sum(e, axis=1, keepdims=True)
    idx_ref[...] = ids.astype(jnp.int32)


def _topk(qf, kt3, n):
    av, ai = pl.pallas_call(
        _topk_body,
        grid=(n // BQ,),
        in_specs=[
            pl.BlockSpec((BQ, CF), lambda i: (i, 0)),
            pl.BlockSpec((NT, CF, CW), lambda i: (0, 0, 0)),
        ],
        out_specs=[
            pl.BlockSpec((BQ, K * AW), lambda i: (i, 0)),
            pl.BlockSpec((BQ, K * AW), lambda i: (i, 0)),
        ],
        out_shape=[
            jax.ShapeDtypeStruct((n, K * AW), jnp.float32),
            jax.ShapeDtypeStruct((n, K * AW), jnp.float32),
        ],
        scratch_shapes=[pltpu.VMEM((N // AW, BQ, AW), jnp.float32)],
    )(qf, kt3)
    return pl.pallas_call(
        _merge_body,
        grid=(n // BQM,),
        in_specs=[
            pl.BlockSpec((BQM, K * AW), lambda i: (i, 0)),
            pl.BlockSpec((BQM, K * AW), lambda i: (i, 0)),
        ],
        out_specs=[
            pl.BlockSpec((BQM, K), lambda i: (i, 0)),
            pl.BlockSpec((BQM, K), lambda i: (i, 0)),
        ],
        out_shape=[
            jax.ShapeDtypeStruct((n, K), jnp.float32),
            jax.ShapeDtypeStruct((n, K), jnp.int32),
        ],
    )(av, ai)


def _sc_att(vT, idx, wgt, n):
    NI = n * K
    QW = 32
    VD = 128
    vpad = jnp.pad(vT, ((0, 0), (0, VD - CF)))
    wB = jnp.broadcast_to(wgt.reshape(n, K, 1), (n, K, CF)).reshape(n, K * CF)
    mesh = plsc.VectorSubcoreMesh(core_axis_name="core",
                                  subcore_axis_name="subcore")

    @pl.kernel(out_type=jax.ShapeDtypeStruct((n // 8, VD), jnp.float32),
               mesh=mesh,
               scratch_types=[pltpu.VMEM((QW * K, VD), jnp.float32)])
    def gk(v_hbm, i_hbm, w_hbm, o_hbm, g_scr):
        def body(i_vmem, w_vmem, o_vmem):
            pltpu.sync_copy(v_hbm.at[i_vmem.at[0]], g_scr)

            @pl.loop(0, QW // 8)
            def _(q0):
                for qq in range(8):
                    q = q0 * 8 + qq
                    acc = w_vmem[q, 0:CF] * g_scr[q * K, 0:CF]
                    for k in range(1, K):
                        acc = acc + (w_vmem[q, k * CF:(k + 1) * CF]
                                     * g_scr[q * K + k, 0:CF])
                    o_vmem[q0, qq * CF:(qq + 1) * CF] = acc

        pltpu.emit_pipeline(
            body,
            grid=(NI // (QW * K),),
            in_specs=[pl.BlockSpec((1, QW * K), index_map=lambda i: (0, i)),
                      pl.BlockSpec((QW, VD), index_map=lambda i: (i, 0))],
            out_specs=[pl.BlockSpec((QW // 8, VD), index_map=lambda i: (i, 0))],
            core_axis_name='subcore',
            dimension_semantics=(pltpu.PARALLEL,),
        )(i_hbm, w_hbm, o_hbm)

    out = gk(vpad, idx.reshape(1, NI), wB)
    return out.reshape(n, CF)


def kernel(a, b, Wq, bq, Wk, bk, Wv, bv, Wf, bf):
    q = jax.nn.relu(_conv(a, Wq, bq))[0]
    k = jax.nn.relu(_conv(b, Wk, bk))[0]
    v = jax.nn.relu(_conv(b, Wv, bv))[0]

    qf = q.reshape(CF, N).T
    kt3 = k.reshape(CF, NT, CW).transpose(1, 0, 2)
    vT = v.reshape(CF, N).T

    NS = 2
    n = N // NS
    att_parts = []
    for p in range(NS):
        wgt, idx = _topk(qf[p * n:(p + 1) * n], kt3, n)
        att_parts.append(_sc_att(vT, idx, wgt, n))
    att = jnp.concatenate(att_parts, axis=0)
    att = att.T.reshape(1, CF, H, W)

    out = jax.nn.sigmoid(_conv(jnp.concatenate([a, att], axis=1), Wf, bf))
    return out


# speedup vs baseline: 1.4208x; 1.0384x over previous
import jax
import jax.numpy as jnp
from jax.experimental import pallas as pl
from jax.experimental.pallas import tpu as pltpu
from jax.experimental.pallas import tpu_sc as plsc

H = 128
W = 128
CF = 16
K = 8
N = H * W
BQ = 128
SQ = 16
CW = 1024
NT = N // CW


def _conv(x, w, b):
    y = jax.lax.conv_general_dilated(
        x, w, (1, 1), 'SAME', dimension_numbers=('NCHW', 'OIHW', 'NCHW'))
    return y + b[None, :, None, None]


AW = 128
NSUB = CW // AW


def _topk_body(q_ref, kt_ref, av_ref, ai_ref, cost_scr):
    q = q_ref[...]
    q2 = jnp.sum(q * q, axis=1, keepdims=True)

    inf = jnp.float32(jnp.inf)
    lane = jax.lax.broadcasted_iota(
        jnp.int32, (SQ, AW), 1).astype(jnp.float32)

    def mm_step(t, carry):
        kt = kt_ref[t]
        k2 = jnp.sum(kt * kt, axis=0, keepdims=True)
        c = q2 - 2.0 * jnp.dot(q, kt, preferred_element_type=jnp.float32) + k2
        for s in range(NSUB):
            cost_scr[t * NSUB + s] = c[:, s * AW:(s + 1) * AW]
        return carry

    jax.lax.fori_loop(0, NT, mm_step, 0, unroll=2)

    for qb in range(BQ // SQ):
        acc_v0 = tuple(jnp.full((SQ, AW), inf, jnp.float32)
                       for _ in range(K))
        acc_i0 = tuple(jnp.full((SQ, AW), 2.0 ** 30, jnp.float32)
                       for _ in range(K))

        def ins_step(u, carry):
            av, ai = carry
            av = list(av)
            ai = list(ai)
            x = cost_scr[u, qb * SQ:(qb + 1) * SQ, :]
            gx = lane + (u * AW).astype(jnp.float32)
            m = [x < av[j] for j in range(K)]
            for j in range(K - 1, 0, -1):
                av[j] = jnp.where(m[j - 1], av[j - 1],
                                  jnp.where(m[j], x, av[j]))
                ai[j] = jnp.where(m[j - 1], ai[j - 1],
                                  jnp.where(m[j], gx, ai[j]))
            av[0] = jnp.where(m[0], x, av[0])
            ai[0] = jnp.where(m[0], gx, ai[0])
            return tuple(av), tuple(ai)

        av, ai = jax.lax.fori_loop(0, NT * NSUB, ins_step,
                                   (acc_v0, acc_i0), unroll=2)
        av_ref[qb * SQ:(qb + 1) * SQ, :] = jnp.concatenate(av, axis=1)
        ai_ref[qb * SQ:(qb + 1) * SQ, :] = jnp.concatenate(ai, axis=1)


BQM = 256


def _merge_body(av_ref, ai_ref, wgt_ref, idx_ref):
    fv = av_ref[...]
    fi = ai_ref[...]
    inf = jnp.float32(jnp.inf)
    ms = []
    idxs = []
    m_prev = jnp.full((BQM, 1), -inf, dtype=jnp.float32)
    i_prev = jnp.full((BQM, 1), -1.0, dtype=jnp.float32)
    for k in range(K):
        valid = (fv > m_prev) | ((fv == m_prev) & (fi > i_prev))
        ceff = jnp.where(valid, fv, inf)
        m = jnp.min(ceff, axis=1, keepdims=True)
        i = jnp.min(jnp.where(ceff == m, fi, jnp.float32(2.0 ** 30)),
                    axis=1, keepdims=True)
        ms.append(m)
        idxs.append(i)
        m_prev, i_prev = m, i

    costs = jnp.concatenate(ms, axis=1)
    ids = jnp.concatenate(idxs, axis=1)
    e = jnp.exp(costs[:, 0:1] - costs)
    wgt_ref[...] = e / jnp.sum(e, axis=1, keepdims=True)
    idx_ref[...] = ids.astype(jnp.int32)


def _topk(qf, kt3, n):
    av, ai = pl.pallas_call(
        _topk_body,
        grid=(n // BQ,),
        in_specs=[
            pl.BlockSpec((BQ, CF), lambda i: (i, 0)),
            pl.BlockSpec((NT, CF, CW), lambda i: (0, 0, 0)),
        ],
        out_specs=[
            pl.BlockSpec((BQ, K * AW), lambda i: (i, 0)),
            pl.BlockSpec((BQ, K * AW), lambda i: (i, 0)),
        ],
        out_shape=[
            jax.ShapeDtypeStruct((n, K * AW), jnp.float32),
            jax.ShapeDtypeStruct((n, K * AW), jnp.float32),
        ],
        scratch_shapes=[pltpu.VMEM((N // AW, BQ, AW), jnp.float32)],
    )(qf, kt3)
    return pl.pallas_call(
        _merge_body,
        grid=(n // BQM,),
        in_specs=[
            pl.BlockSpec((BQM, K * AW), lambda i: (i, 0)),
            pl.BlockSpec((BQM, K * AW), lambda i: (i, 0)),
        ],
        out_specs=[
            pl.BlockSpec((BQM, K), lambda i: (i, 0)),
            pl.BlockSpec((BQM, K), lambda i: (i, 0)),
        ],
        out_shape=[
            jax.ShapeDtypeStruct((n, K), jnp.float32),
            jax.ShapeDtypeStruct((n, K), jnp.int32),
        ],
    )(av, ai)


def _sc_att(vT, idx, wgt, n):
    NI = n * K
    QW = 32
    VD = 128
    vpad = jnp.pad(vT, ((0, 0), (0, VD - CF)))
    wB = jnp.broadcast_to(wgt.reshape(n, K, 1), (n, K, CF)).reshape(n, K * CF)
    mesh = plsc.VectorSubcoreMesh(core_axis_name="core",
                                  subcore_axis_name="subcore")

    @pl.kernel(out_type=jax.ShapeDtypeStruct((n // 8, VD), jnp.float32),
               mesh=mesh,
               scratch_types=[pltpu.VMEM((QW * K, VD), jnp.float32)])
    def gk(v_hbm, i_hbm, w_hbm, o_hbm, g_scr):
        def body(i_vmem, w_vmem, o_vmem):
            pltpu.sync_copy(v_hbm.at[i_vmem.at[0]], g_scr)

            @pl.loop(0, QW // 8)
            def _(q0):
                for qq in range(8):
                    q = q0 * 8 + qq
                    acc = w_vmem[q, 0:CF] * g_scr[q * K, 0:CF]
                    for k in range(1, K):
                        acc = acc + (w_vmem[q, k * CF:(k + 1) * CF]
                                     * g_scr[q * K + k, 0:CF])
                    o_vmem[q0, qq * CF:(qq + 1) * CF] = acc

        pltpu.emit_pipeline(
            body,
            grid=(NI // (QW * K),),
            in_specs=[pl.BlockSpec((1, QW * K), index_map=lambda i: (0, i)),
                      pl.BlockSpec((QW, VD), index_map=lambda i: (i, 0))],
            out_specs=[pl.BlockSpec((QW // 8, VD), index_map=lambda i: (i, 0))],
            core_axis_name='subcore',
            dimension_semantics=(pltpu.PARALLEL,),
        )(i_hbm, w_hbm, o_hbm)

    out = gk(vpad, idx.reshape(1, NI), wB)
    return out.reshape(n, CF)


def kernel(a, b, Wq, bq, Wk, bk, Wv, bv, Wf, bf):
    q = jax.nn.relu(_conv(a, Wq, bq))[0]
    k = jax.nn.relu(_conv(b, Wk, bk))[0]
    v = jax.nn.relu(_conv(b, Wv, bv))[0]

    qf = q.reshape(CF, N).T
    kt3 = k.reshape(CF, NT, CW).transpose(1, 0, 2)
    vT = v.reshape(CF, N).T

    NS = 4
    n = N // NS
    att_parts = []
    for p in range(NS):
        wgt, idx = _topk(qf[p * n:(p + 1) * n], kt3, n)
        att_parts.append(_sc_att(vT, idx, wgt, n))
    att = jnp.concatenate(att_parts, axis=0)
    att = att.T.reshape(1, CF, H, W)

    out = jax.nn.sigmoid(_conv(jnp.concatenate([a, att], axis=1), Wf, bf))
    return out


# speedup vs baseline: 1.4238x; 1.0021x over previous
import jax
import jax.numpy as jnp
from jax.experimental import pallas as pl
from jax.experimental.pallas import tpu as pltpu
from jax.experimental.pallas import tpu_sc as plsc

H = 128
W = 128
CF = 16
K = 8
N = H * W
BQ = 128
SQ = 16
CW = 1024
NT = N // CW


def _conv(x, w, b):
    y = jax.lax.conv_general_dilated(
        x, w, (1, 1), 'SAME', dimension_numbers=('NCHW', 'OIHW', 'NCHW'))
    return y + b[None, :, None, None]


AW = 128
NSUB = CW // AW


def _topk_body(q_ref, kt_ref, av_ref, ai_ref, cost_scr):
    q = q_ref[...]
    q2 = jnp.sum(q * q, axis=1, keepdims=True)

    inf = jnp.float32(jnp.inf)
    lane = jax.lax.broadcasted_iota(
        jnp.int32, (SQ, AW), 1).astype(jnp.float32)

    def mm_step(t, carry):
        for s in range(NSUB):
            kts = kt_ref[t, :, s * AW:(s + 1) * AW]
            k2 = jnp.sum(kts * kts, axis=0, keepdims=True)
            cost_scr[t * NSUB + s] = (
                q2 - 2.0 * jnp.dot(q, kts,
                                   preferred_element_type=jnp.float32) + k2)
        return carry

    jax.lax.fori_loop(0, NT, mm_step, 0, unroll=2)

    for qb in range(BQ // SQ):
        acc_v0 = tuple(jnp.full((SQ, AW), inf, jnp.float32)
                       for _ in range(K))
        acc_i0 = tuple(jnp.full((SQ, AW), 2.0 ** 30, jnp.float32)
                       for _ in range(K))

        def ins_step(u, carry):
            av, ai = carry
            av = list(av)
            ai = list(ai)
            x = cost_scr[u, qb * SQ:(qb + 1) * SQ, :]
            gx = lane + (u * AW).astype(jnp.float32)
            m = [x < av[j] for j in range(K)]
            for j in range(K - 1, 0, -1):
                av[j] = jnp.where(m[j - 1], av[j - 1],
                                  jnp.where(m[j], x, av[j]))
                ai[j] = jnp.where(m[j - 1], ai[j - 1],
                                  jnp.where(m[j], gx, ai[j]))
            av[0] = jnp.where(m[0], x, av[0])
            ai[0] = jnp.where(m[0], gx, ai[0])
            return tuple(av), tuple(ai)

        av, ai = jax.lax.fori_loop(0, NT * NSUB, ins_step,
                                   (acc_v0, acc_i0), unroll=2)
        av_ref[qb * SQ:(qb + 1) * SQ, :] = jnp.concatenate(av, axis=1)
        ai_ref[qb * SQ:(qb + 1) * SQ, :] = jnp.concatenate(ai, axis=1)


BQM = 256


def _merge_body(av_ref, ai_ref, wgt_ref, idx_ref):
    fv = av_ref[...]
    fi = ai_ref[...]
    inf = jnp.float32(jnp.inf)
    ms = []
    idxs = []
    m_prev = jnp.full((BQM, 1), -inf, dtype=jnp.float32)
    i_prev = jnp.full((BQM, 1), -1.0, dtype=jnp.float32)
    for k in range(K):
        valid = (fv > m_prev) | ((fv == m_prev) & (fi > i_prev))
        ceff = jnp.where(valid, fv, inf)
        m = jnp.min(ceff, axis=1, keepdims=True)
        i = jnp.min(jnp.where(ceff == m, fi, jnp.float32(2.0 ** 30)),
                    axis=1, keepdims=True)
        ms.append(m)
        idxs.append(i)
        m_prev, i_prev = m, i

    costs = jnp.concatenate(ms, axis=1)
    ids = jnp.concatenate(idxs, axis=1)
    e = jnp.exp(costs[:, 0:1] - costs)
    wgt_ref[...] = e / jnp.sum(e, axis=1, keepdims=True)
    idx_ref[...] = ids.astype(jnp.int32)


def _topk(qf, kt3, n):
    av, ai = pl.pallas_call(
        _topk_body,
        grid=(n // BQ,),
        in_specs=[
            pl.BlockSpec((BQ, CF), lambda i: (i, 0)),
            pl.BlockSpec((NT, CF, CW), lambda i: (0, 0, 0)),
        ],
        out_specs=[
            pl.BlockSpec((BQ, K * AW), lambda i: (i, 0)),
            pl.BlockSpec((BQ, K * AW), lambda i: (i, 0)),
        ],
        out_shape=[
            jax.ShapeDtypeStruct((n, K * AW), jnp.float32),
            jax.ShapeDtypeStruct((n, K * AW), jnp.float32),
        ],
        scratch_shapes=[pltpu.VMEM((N // AW, BQ, AW), jnp.float32)],
    )(qf, kt3)
    return pl.pallas_call(
        _merge_body,
        grid=(n // BQM,),
        in_specs=[
            pl.BlockSpec((BQM, K * AW), lambda i: (i, 0)),
            pl.BlockSpec((BQM, K * AW), lambda i: (i, 0)),
        ],
        out_specs=[
            pl.BlockSpec((BQM, K), lambda i: (i, 0)),
            pl.BlockSpec((BQM, K), lambda i: (i, 0)),
        ],
        out_shape=[
            jax.ShapeDtypeStruct((n, K), jnp.float32),
            jax.ShapeDtypeStruct((n, K), jnp.int32),
        ],
    )(av, ai)


def _sc_att(vT, idx, wgt, n):
    NI = n * K
    QW = 32
    VD = 128
    vpad = jnp.pad(vT, ((0, 0), (0, VD - CF)))
    wB = jnp.broadcast_to(wgt.reshape(n, K, 1), (n, K, CF)).reshape(n, K * CF)
    mesh = plsc.VectorSubcoreMesh(core_axis_name="core",
                                  subcore_axis_name="subcore")

    @pl.kernel(out_type=jax.ShapeDtypeStruct((n // 8, VD), jnp.float32),
               mesh=mesh,
               scratch_types=[pltpu.VMEM((QW * K, VD), jnp.float32)])
    def gk(v_hbm, i_hbm, w_hbm, o_hbm, g_scr):
        def body(i_vmem, w_vmem, o_vmem):
            pltpu.sync_copy(v_hbm.at[i_vmem.at[0]], g_scr)

            @pl.loop(0, QW // 8)
            def _(q0):
                for qq in range(8):
                    q = q0 * 8 + qq
                    acc = w_vmem[q, 0:CF] * g_scr[q * K, 0:CF]
                    for k in range(1, K):
                        acc = acc + (w_vmem[q, k * CF:(k + 1) * CF]
                                     * g_scr[q * K + k, 0:CF])
                    o_vmem[q0, qq * CF:(qq + 1) * CF] = acc

        pltpu.emit_pipeline(
            body,
            grid=(NI // (QW * K),),
            in_specs=[pl.BlockSpec((1, QW * K), index_map=lambda i: (0, i)),
                      pl.BlockSpec((QW, VD), index_map=lambda i: (i, 0))],
            out_specs=[pl.BlockSpec((QW // 8, VD), index_map=lambda i: (i, 0))],
            core_axis_name='subcore',
            dimension_semantics=(pltpu.PARALLEL,),
        )(i_hbm, w_hbm, o_hbm)

    out = gk(vpad, idx.reshape(1, NI), wB)
    return out.reshape(n, CF)


def kernel(a, b, Wq, bq, Wk, bk, Wv, bv, Wf, bf):
    q = jax.nn.relu(_conv(a, Wq, bq))[0]
    k = jax.nn.relu(_conv(b, Wk, bk))[0]
    v = jax.nn.relu(_conv(b, Wv, bv))[0]

    qf = q.reshape(CF, N).T
    kt3 = k.reshape(CF, NT, CW).transpose(1, 0, 2)
    vT = v.reshape(CF, N).T

    NS = 4
    n = N // NS
    att_parts = []
    for p in range(NS):
        wgt, idx = _topk(qf[p * n:(p + 1) * n], kt3, n)
        att_parts.append(_sc_att(vT, idx, wgt, n))
    att = jnp.concatenate(att_parts, axis=0)
    att = att.T.reshape(1, CF, H, W)

    out = jax.nn.sigmoid(_conv(jnp.concatenate([a, att], axis=1), Wf, bf))
    return out


# speedup vs baseline: 1.5727x; 1.1046x over previous
import jax
import jax.numpy as jnp
from jax.experimental import pallas as pl
from jax.experimental.pallas import tpu as pltpu
from jax.experimental.pallas import tpu_sc as plsc

H = 128
W = 128
CF = 16
K = 8
N = H * W
BQ = 128
SQ = 16
CW = 1024
NT = N // CW


def _conv(x, w, b):
    y = jax.lax.conv_general_dilated(
        x, w, (1, 1), 'SAME', dimension_numbers=('NCHW', 'OIHW', 'NCHW'))
    return y + b[None, :, None, None]


AW = 128
NSUB = CW // AW


def _topk_body(q_ref, kt_ref, av_ref, ai_ref, cost_scr):
    q = q_ref[...]
    q2 = jnp.sum(q * q, axis=1, keepdims=True)

    inf = jnp.float32(jnp.inf)
    lane = jax.lax.broadcasted_iota(
        jnp.int32, (SQ, AW), 1).astype(jnp.float32)

    def mm_step(t, carry):
        for s in range(NSUB):
            kts = kt_ref[t, :, s * AW:(s + 1) * AW]
            k2 = jnp.sum(kts * kts, axis=0, keepdims=True)
            cost_scr[t * NSUB + s] = (
                q2 - 2.0 * jnp.dot(q, kts,
                                   preferred_element_type=jnp.float32) + k2)
        return carry

    jax.lax.fori_loop(0, NT, mm_step, 0, unroll=2)

    for qb in range(BQ // SQ):
        acc_v0 = tuple(jnp.full((SQ, AW), inf, jnp.float32)
                       for _ in range(K))
        acc_i0 = tuple(jnp.full((SQ, AW), 2.0 ** 30, jnp.float32)
                       for _ in range(K))

        def ins_step(u, carry):
            av, ai = carry
            av = list(av)
            ai = list(ai)
            x = cost_scr[u, qb * SQ:(qb + 1) * SQ, :]
            gx = lane + (u * AW).astype(jnp.float32)
            m = [x < av[j] for j in range(K)]
            for j in range(K - 1, 0, -1):
                av[j] = jnp.where(m[j - 1], av[j - 1],
                                  jnp.where(m[j], x, av[j]))
                ai[j] = jnp.where(m[j - 1], ai[j - 1],
                                  jnp.where(m[j], gx, ai[j]))
            av[0] = jnp.where(m[0], x, av[0])
            ai[0] = jnp.where(m[0], gx, ai[0])
            return tuple(av), tuple(ai)

        av, ai = jax.lax.fori_loop(0, NT * NSUB, ins_step,
                                   (acc_v0, acc_i0), unroll=4)
        av_ref[qb * SQ:(qb + 1) * SQ, :] = jnp.concatenate(av, axis=1)
        ai_ref[qb * SQ:(qb + 1) * SQ, :] = jnp.concatenate(ai, axis=1)


BQM = 256


def _merge_body(av_ref, ai_ref, wgt_ref, idx_ref):
    fv = av_ref[...]
    fi = ai_ref[...]
    inf = jnp.float32(jnp.inf)
    ms = []
    idxs = []
    m_prev = jnp.full((BQM, 1), -inf, dtype=jnp.float32)
    i_prev = jnp.full((BQM, 1), -1.0, dtype=jnp.float32)
    for k in range(K):
        valid = (fv > m_prev) | ((fv == m_prev) & (fi > i_prev))
        ceff = jnp.where(valid, fv, inf)
        m = jnp.min(ceff, axis=1, keepdims=True)
        i = jnp.min(jnp.where(ceff == m, fi, jnp.float32(2.0 ** 30)),
                    axis=1, keepdims=True)
        ms.append(m)
        idxs.append(i)
        m_prev, i_prev = m, i

    costs = jnp.concatenate(ms, axis=1)
    ids = jnp.concatenate(idxs, axis=1)
    e = jnp.exp(costs[:, 0:1] - costs)
    wgt_ref[...] = e / jnp.sum(e, axis=1, keepdims=True)
    idx_ref[...] = ids.astype(jnp.int32)


def _topk(qf, kt3, n):
    av, ai = pl.pallas_call(
        _topk_body,
        grid=(n // BQ,),
        in_specs=[
            pl.BlockSpec((BQ, CF), lambda i: (i, 0)),
            pl.BlockSpec((NT, CF, CW), lambda i: (0, 0, 0)),
        ],
        out_specs=[
            pl.BlockSpec((BQ, K * AW), lambda i: (i, 0)),
            pl.BlockSpec((BQ, K * AW), lambda i: (i, 0)),
        ],
        out_shape=[
            jax.ShapeDtypeStruct((n, K * AW), jnp.float32),
            jax.ShapeDtypeStruct((n, K * AW), jnp.float32),
        ],
        scratch_shapes=[pltpu.VMEM((N // AW, BQ, AW), jnp.float32)],
    )(qf, kt3)
    return pl.pallas_call(
        _merge_body,
        grid=(n // BQM,),
        in_specs=[
            pl.BlockSpec((BQM, K * AW), lambda i: (i, 0)),
            pl.BlockSpec((BQM, K * AW), lambda i: (i, 0)),
        ],
        out_specs=[
            pl.BlockSpec((BQM, K), lambda i: (i, 0)),
            pl.BlockSpec((BQM, K), lambda i: (i, 0)),
        ],
        out_shape=[
            jax.ShapeDtypeStruct((n, K), jnp.float32),
            jax.ShapeDtypeStruct((n, K), jnp.int32),
        ],
    )(av, ai)


def _sc_att(vT, idx, wgt, n):
    NI = n * K
    QW = 32
    VD = 128
    vpad = jnp.pad(vT, ((0, 0), (0, VD - CF)))
    wB = jnp.broadcast_to(wgt.reshape(n, K, 1), (n, K, CF)).reshape(n, K * CF)
    mesh = plsc.VectorSubcoreMesh(core_axis_name="core",
                                  subcore_axis_name="subcore")

    @pl.kernel(out_type=jax.ShapeDtypeStruct((n // 8, VD), jnp.float32),
               mesh=mesh,
               scratch_types=[pltpu.VMEM((QW * K, VD), jnp.float32)])
    def gk(v_hbm, i_hbm, w_hbm, o_hbm, g_scr):
        def body(i_vmem, w_vmem, o_vmem):
            pltpu.sync_copy(v_hbm.at[i_vmem.at[0]], g_scr)

            @pl.loop(0, QW // 8)
            def _(q0):
                for qq in range(8):
                    q = q0 * 8 + qq
                    acc = w_vmem[q, 0:CF] * g_scr[q * K, 0:CF]
                    for k in range(1, K):
                        acc = acc + (w_vmem[q, k * CF:(k + 1) * CF]
                                     * g_scr[q * K + k, 0:CF])
                    o_vmem[q0, qq * CF:(qq + 1) * CF] = acc

        pltpu.emit_pipeline(
            body,
            grid=(NI // (QW * K),),
            in_specs=[pl.BlockSpec((1, QW * K), index_map=lambda i: (0, i)),
                      pl.BlockSpec((QW, VD), index_map=lambda i: (i, 0))],
            out_specs=[pl.BlockSpec((QW // 8, VD), index_map=lambda i: (i, 0))],
            core_axis_name='subcore',
            dimension_semantics=(pltpu.PARALLEL,),
        )(i_hbm, w_hbm, o_hbm)

    out = gk(vpad, idx.reshape(1, NI), wB)
    return out.reshape(n, CF)


def kernel(a, b, Wq, bq, Wk, bk, Wv, bv, Wf, bf):
    q = jax.nn.relu(_conv(a, Wq, bq))[0]
    k = jax.nn.relu(_conv(b, Wk, bk))[0]
    v = jax.nn.relu(_conv(b, Wv, bv))[0]

    qf = q.reshape(CF, N).T
    kt3 = k.reshape(CF, NT, CW).transpose(1, 0, 2)
    vT = v.reshape(CF, N).T

    NS = 4
    n = N // NS
    att_parts = []
    for p in range(NS):
        wgt, idx = _topk(qf[p * n:(p + 1) * n], kt3, n)
        att_parts.append(_sc_att(vT, idx, wgt, n))
    att = jnp.concatenate(att_parts, axis=0)
    att = att.T.reshape(1, CF, H, W)

    out = jax.nn.sigmoid(_conv(jnp.concatenate([a, att], axis=1), Wf, bf))
    return out


# speedup vs baseline: 1.6541x; 1.0518x over previous
import jax
import jax.numpy as jnp
from jax.experimental import pallas as pl
from jax.experimental.pallas import tpu as pltpu
from jax.experimental.pallas import tpu_sc as plsc

H = 128
W = 128
CF = 16
K = 8
N = H * W
BQ = 128
SQ = 16
CW = 1024
NT = N // CW


def _conv(x, w, b):
    y = jax.lax.conv_general_dilated(
        x, w, (1, 1), 'SAME', dimension_numbers=('NCHW', 'OIHW', 'NCHW'))
    return y + b[None, :, None, None]


AW = 128
NSUB = CW // AW


def _topk_body(q_ref, kt_ref, av_ref, ai_ref, cost_scr):
    q = q_ref[...]
    q2 = jnp.sum(q * q, axis=1, keepdims=True)

    inf = jnp.float32(jnp.inf)
    lane = jax.lax.broadcasted_iota(
        jnp.int32, (SQ, AW), 1).astype(jnp.float32)

    def mm_step(t, carry):
        for s in range(NSUB):
            kts = kt_ref[t, :, s * AW:(s + 1) * AW]
            k2 = jnp.sum(kts * kts, axis=0, keepdims=True)
            cost_scr[t * NSUB + s] = (
                q2 - 2.0 * jnp.dot(q, kts,
                                   preferred_element_type=jnp.float32) + k2)
        return carry

    jax.lax.fori_loop(0, NT, mm_step, 0, unroll=2)

    for qb in range(BQ // SQ):
        acc_v0 = tuple(jnp.full((SQ, AW), inf, jnp.float32)
                       for _ in range(K))
        acc_i0 = tuple(jnp.full((SQ, AW), 2.0 ** 30, jnp.float32)
                       for _ in range(K))

        def ins_step(u, carry):
            av, ai = carry
            av = list(av)
            ai = list(ai)
            x = cost_scr[u, qb * SQ:(qb + 1) * SQ, :]
            gx = lane + (u * AW).astype(jnp.float32)
            m = [x < av[j] for j in range(K)]
            for j in range(K - 1, 0, -1):
                av[j] = jnp.where(m[j - 1], av[j - 1],
                                  jnp.where(m[j], x, av[j]))
                ai[j] = jnp.where(m[j - 1], ai[j - 1],
                                  jnp.where(m[j], gx, ai[j]))
            av[0] = jnp.where(m[0], x, av[0])
            ai[0] = jnp.where(m[0], gx, ai[0])
            return tuple(av), tuple(ai)

        av, ai = jax.lax.fori_loop(0, NT * NSUB, ins_step,
                                   (acc_v0, acc_i0), unroll=8)
        av_ref[qb * SQ:(qb + 1) * SQ, :] = jnp.concatenate(av, axis=1)
        ai_ref[qb * SQ:(qb + 1) * SQ, :] = jnp.concatenate(ai, axis=1)


BQM = 256


def _merge_body(av_ref, ai_ref, wgt_ref, idx_ref):
    fv = av_ref[...]
    fi = ai_ref[...]
    inf = jnp.float32(jnp.inf)
    ms = []
    idxs = []
    m_prev = jnp.full((BQM, 1), -inf, dtype=jnp.float32)
    i_prev = jnp.full((BQM, 1), -1.0, dtype=jnp.float32)
    for k in range(K):
        valid = (fv > m_prev) | ((fv == m_prev) & (fi > i_prev))
        ceff = jnp.where(valid, fv, inf)
        m = jnp.min(ceff, axis=1, keepdims=True)
        i = jnp.min(jnp.where(ceff == m, fi, jnp.float32(2.0 ** 30)),
                    axis=1, keepdims=True)
        ms.append(m)
        idxs.append(i)
        m_prev, i_prev = m, i

    costs = jnp.concatenate(ms, axis=1)
    ids = jnp.concatenate(idxs, axis=1)
    e = jnp.exp(costs[:, 0:1] - costs)
    wgt_ref[...] = e / jnp.sum(e, axis=1, keepdims=True)
    idx_ref[...] = ids.astype(jnp.int32)


def _topk(qf, kt3, n):
    av, ai = pl.pallas_call(
        _topk_body,
        grid=(n // BQ,),
        in_specs=[
            pl.BlockSpec((BQ, CF), lambda i: (i, 0)),
            pl.BlockSpec((NT, CF, CW), lambda i: (0, 0, 0)),
        ],
        out_specs=[
            pl.BlockSpec((BQ, K * AW), lambda i: (i, 0)),
            pl.BlockSpec((BQ, K * AW), lambda i: (i, 0)),
        ],
        out_shape=[
            jax.ShapeDtypeStruct((n, K * AW), jnp.float32),
            jax.ShapeDtypeStruct((n, K * AW), jnp.float32),
        ],
        scratch_shapes=[pltpu.VMEM((N // AW, BQ, AW), jnp.float32)],
    )(qf, kt3)
    return pl.pallas_call(
        _merge_body,
        grid=(n // BQM,),
        in_specs=[
            pl.BlockSpec((BQM, K * AW), lambda i: (i, 0)),
            pl.BlockSpec((BQM, K * AW), lambda i: (i, 0)),
        ],
        out_specs=[
            pl.BlockSpec((BQM, K), lambda i: (i, 0)),
            pl.BlockSpec((BQM, K), lambda i: (i, 0)),
        ],
        out_shape=[
            jax.ShapeDtypeStruct((n, K), jnp.float32),
            jax.ShapeDtypeStruct((n, K), jnp.int32),
        ],
    )(av, ai)


def _sc_att(vT, idx, wgt, n):
    NI = n * K
    QW = 32
    VD = 128
    vpad = jnp.pad(vT, ((0, 0), (0, VD - CF)))
    wB = jnp.broadcast_to(wgt.reshape(n, K, 1), (n, K, CF)).reshape(n, K * CF)
    mesh = plsc.VectorSubcoreMesh(core_axis_name="core",
                                  subcore_axis_name="subcore")

    @pl.kernel(out_type=jax.ShapeDtypeStruct((n // 8, VD), jnp.float32),
               mesh=mesh,
               scratch_types=[pltpu.VMEM((QW * K, VD), jnp.float32)])
    def gk(v_hbm, i_hbm, w_hbm, o_hbm, g_scr):
        def body(i_vmem, w_vmem, o_vmem):
            pltpu.sync_copy(v_hbm.at[i_vmem.at[0]], g_scr)

            @pl.loop(0, QW // 8)
            def _(q0):
                for qq in range(8):
                    q = q0 * 8 + qq
                    acc = w_vmem[q, 0:CF] * g_scr[q * K, 0:CF]
                    for k in range(1, K):
                        acc = acc + (w_vmem[q, k * CF:(k + 1) * CF]
                                     * g_scr[q * K + k, 0:CF])
                    o_vmem[q0, qq * CF:(qq + 1) * CF] = acc

        pltpu.emit_pipeline(
            body,
            grid=(NI // (QW * K),),
            in_specs=[pl.BlockSpec((1, QW * K), index_map=lambda i: (0, i)),
                      pl.BlockSpec((QW, VD), index_map=lambda i: (i, 0))],
            out_specs=[pl.BlockSpec((QW // 8, VD), index_map=lambda i: (i, 0))],
            core_axis_name='subcore',
            dimension_semantics=(pltpu.PARALLEL,),
        )(i_hbm, w_hbm, o_hbm)

    out = gk(vpad, idx.reshape(1, NI), wB)
    return out.reshape(n, CF)


def kernel(a, b, Wq, bq, Wk, bk, Wv, bv, Wf, bf):
    q = jax.nn.relu(_conv(a, Wq, bq))[0]
    k = jax.nn.relu(_conv(b, Wk, bk))[0]
    v = jax.nn.relu(_conv(b, Wv, bv))[0]

    qf = q.reshape(CF, N).T
    kt3 = k.reshape(CF, NT, CW).transpose(1, 0, 2)
    vT = v.reshape(CF, N).T

    NS = 4
    n = N // NS
    att_parts = []
    for p in range(NS):
        wgt, idx = _topk(qf[p * n:(p + 1) * n], kt3, n)
        att_parts.append(_sc_att(vT, idx, wgt, n))
    att = jnp.concatenate(att_parts, axis=0)
    att = att.T.reshape(1, CF, H, W)

    out = jax.nn.sigmoid(_conv(jnp.concatenate([a, att], axis=1), Wf, bf))
    return out


# speedup vs baseline: 1.7520x; 1.0592x over previous
import jax
import jax.numpy as jnp
from jax.experimental import pallas as pl
from jax.experimental.pallas import tpu as pltpu
from jax.experimental.pallas import tpu_sc as plsc

H = 128
W = 128
CF = 16
K = 8
N = H * W
BQ = 128
SQ = 16
CW = 1024
NT = N // CW


def _conv(x, w, b):
    y = jax.lax.conv_general_dilated(
        x, w, (1, 1), 'SAME', dimension_numbers=('NCHW', 'OIHW', 'NCHW'))
    return y + b[None, :, None, None]


AW = 128
NSUB = CW // AW


def _topk_body(q_ref, kt_ref, av_ref, ai_ref, cost_scr):
    q = q_ref[...]
    q2 = jnp.sum(q * q, axis=1, keepdims=True)

    inf = jnp.float32(jnp.inf)
    lane = jax.lax.broadcasted_iota(
        jnp.int32, (SQ, AW), 1).astype(jnp.float32)

    def mm_step(t, carry):
        for s in range(NSUB):
            kts = kt_ref[t, :, s * AW:(s + 1) * AW]
            k2 = jnp.sum(kts * kts, axis=0, keepdims=True)
            cost_scr[t * NSUB + s] = (
                q2 - 2.0 * jnp.dot(q, kts,
                                   preferred_element_type=jnp.float32) + k2)
        return carry

    jax.lax.fori_loop(0, NT, mm_step, 0, unroll=4)

    for qb in range(BQ // SQ):
        acc_v0 = tuple(jnp.full((SQ, AW), inf, jnp.float32)
                       for _ in range(K))
        acc_i0 = tuple(jnp.full((SQ, AW), 2.0 ** 30, jnp.float32)
                       for _ in range(K))

        def ins_step(u, carry):
            av, ai = carry
            av = list(av)
            ai = list(ai)
            x = cost_scr[u, qb * SQ:(qb + 1) * SQ, :]
            gx = lane + (u * AW).astype(jnp.float32)
            m = [x < av[j] for j in range(K)]
            for j in range(K - 1, 0, -1):
                av[j] = jnp.where(m[j - 1], av[j - 1],
                                  jnp.where(m[j], x, av[j]))
                ai[j] = jnp.where(m[j - 1], ai[j - 1],
                                  jnp.where(m[j], gx, ai[j]))
            av[0] = jnp.where(m[0], x, av[0])
            ai[0] = jnp.where(m[0], gx, ai[0])
            return tuple(av), tuple(ai)

        av, ai = jax.lax.fori_loop(0, NT * NSUB, ins_step,
                                   (acc_v0, acc_i0), unroll=16)
        av_ref[qb * SQ:(qb + 1) * SQ, :] = jnp.concatenate(av, axis=1)
        ai_ref[qb * SQ:(qb + 1) * SQ, :] = jnp.concatenate(ai, axis=1)


BQM = 256


def _merge_body(av_ref, ai_ref, wgt_ref, idx_ref):
    fv = av_ref[...]
    fi = ai_ref[...]
    inf = jnp.float32(jnp.inf)
    ms = []
    idxs = []
    m_prev = jnp.full((BQM, 1), -inf, dtype=jnp.float32)
    i_prev = jnp.full((BQM, 1), -1.0, dtype=jnp.float32)
    for k in range(K):
        valid = (fv > m_prev) | ((fv == m_prev) & (fi > i_prev))
        ceff = jnp.where(valid, fv, inf)
        m = jnp.min(ceff, axis=1, keepdims=True)
        i = jnp.min(jnp.where(ceff == m, fi, jnp.float32(2.0 ** 30)),
                    axis=1, keepdims=True)
        ms.append(m)
        idxs.append(i)
        m_prev, i_prev = m, i

    costs = jnp.concatenate(ms, axis=1)
    ids = jnp.concatenate(idxs, axis=1)
    e = jnp.exp(costs[:, 0:1] - costs)
    wgt_ref[...] = e / jnp.sum(e, axis=1, keepdims=True)
    idx_ref[...] = ids.astype(jnp.int32)


def _topk(qf, kt3, n):
    av, ai = pl.pallas_call(
        _topk_body,
        grid=(n // BQ,),
        in_specs=[
            pl.BlockSpec((BQ, CF), lambda i: (i, 0)),
            pl.BlockSpec((NT, CF, CW), lambda i: (0, 0, 0)),
        ],
        out_specs=[
            pl.BlockSpec((BQ, K * AW), lambda i: (i, 0)),
            pl.BlockSpec((BQ, K * AW), lambda i: (i, 0)),
        ],
        out_shape=[
            jax.ShapeDtypeStruct((n, K * AW), jnp.float32),
            jax.ShapeDtypeStruct((n, K * AW), jnp.float32),
        ],
        scratch_shapes=[pltpu.VMEM((N // AW, BQ, AW), jnp.float32)],
    )(qf, kt3)
    return pl.pallas_call(
        _merge_body,
        grid=(n // BQM,),
        in_specs=[
            pl.BlockSpec((BQM, K * AW), lambda i: (i, 0)),
            pl.BlockSpec((BQM, K * AW), lambda i: (i, 0)),
        ],
        out_specs=[
            pl.BlockSpec((BQM, K), lambda i: (i, 0)),
            pl.BlockSpec((BQM, K), lambda i: (i, 0)),
        ],
        out_shape=[
            jax.ShapeDtypeStruct((n, K), jnp.float32),
            jax.ShapeDtypeStruct((n, K), jnp.int32),
        ],
    )(av, ai)


def _sc_att(vT, idx, wgt, n):
    NI = n * K
    QW = 32
    VD = 128
    vpad = jnp.pad(vT, ((0, 0), (0, VD - CF)))
    wB = jnp.broadcast_to(wgt.reshape(n, K, 1), (n, K, CF)).reshape(n, K * CF)
    mesh = plsc.VectorSubcoreMesh(core_axis_name="core",
                                  subcore_axis_name="subcore")

    @pl.kernel(out_type=jax.ShapeDtypeStruct((n // 8, VD), jnp.float32),
               mesh=mesh,
               scratch_types=[pltpu.VMEM((QW * K, VD), jnp.float32)])
    def gk(v_hbm, i_hbm, w_hbm, o_hbm, g_scr):
        def body(i_vmem, w_vmem, o_vmem):
            pltpu.sync_copy(v_hbm.at[i_vmem.at[0]], g_scr)

            @pl.loop(0, QW // 8)
            def _(q0):
                for qq in range(8):
                    q = q0 * 8 + qq
                    acc = w_vmem[q, 0:CF] * g_scr[q * K, 0:CF]
                    for k in range(1, K):
                        acc = acc + (w_vmem[q, k * CF:(k + 1) * CF]
                                     * g_scr[q * K + k, 0:CF])
                    o_vmem[q0, qq * CF:(qq + 1) * CF] = acc

        pltpu.emit_pipeline(
            body,
            grid=(NI // (QW * K),),
            in_specs=[pl.BlockSpec((1, QW * K), index_map=lambda i: (0, i)),
                      pl.BlockSpec((QW, VD), index_map=lambda i: (i, 0))],
            out_specs=[pl.BlockSpec((QW // 8, VD), index_map=lambda i: (i, 0))],
            core_axis_name='subcore',
            dimension_semantics=(pltpu.PARALLEL,),
        )(i_hbm, w_hbm, o_hbm)

    out = gk(vpad, idx.reshape(1, NI), wB)
    return out.reshape(n, CF)


def kernel(a, b, Wq, bq, Wk, bk, Wv, bv, Wf, bf):
    q = jax.nn.relu(_conv(a, Wq, bq))[0]
    k = jax.nn.relu(_conv(b, Wk, bk))[0]
    v = jax.nn.relu(_conv(b, Wv, bv))[0]

    qf = q.reshape(CF, N).T
    kt3 = k.reshape(CF, NT, CW).transpose(1, 0, 2)
    vT = v.reshape(CF, N).T

    NS = 4
    n = N // NS
    att_parts = []
    for p in range(NS):
        wgt, idx = _topk(qf[p * n:(p + 1) * n], kt3, n)
        att_parts.append(_sc_att(vT, idx, wgt, n))
    att = jnp.concatenate(att_parts, axis=0)
    att = att.T.reshape(1, CF, H, W)

    out = jax.nn.sigmoid(_conv(jnp.concatenate([a, att], axis=1), Wf, bf))
    return out


# speedup vs baseline: 1.8114x; 1.0339x over previous
import jax
import jax.numpy as jnp
from jax.experimental import pallas as pl
from jax.experimental.pallas import tpu as pltpu
from jax.experimental.pallas import tpu_sc as plsc

H = 128
W = 128
CF = 16
K = 8
N = H * W
BQ = 128
SQ = 16
CW = 1024
NT = N // CW


def _conv(x, w, b):
    y = jax.lax.conv_general_dilated(
        x, w, (1, 1), 'SAME', dimension_numbers=('NCHW', 'OIHW', 'NCHW'))
    return y + b[None, :, None, None]


AW = 128
NSUB = CW // AW


def _topk_body(q_ref, kt_ref, av_ref, ai_ref, cost_scr):
    q = q_ref[...]
    q2 = jnp.sum(q * q, axis=1, keepdims=True)

    inf = jnp.float32(jnp.inf)
    lane = jax.lax.broadcasted_iota(
        jnp.int32, (SQ, AW), 1).astype(jnp.float32)

    def mm_step(t, carry):
        for s in range(NSUB):
            kts = kt_ref[t, :, s * AW:(s + 1) * AW]
            k2 = jnp.sum(kts * kts, axis=0, keepdims=True)
            cost_scr[t * NSUB + s] = (
                q2 - 2.0 * jnp.dot(q, kts,
                                   preferred_element_type=jnp.float32) + k2)
        return carry

    jax.lax.fori_loop(0, NT, mm_step, 0, unroll=8)

    for qb in range(BQ // SQ):
        acc_v0 = tuple(jnp.full((SQ, AW), inf, jnp.float32)
                       for _ in range(K))
        acc_i0 = tuple(jnp.full((SQ, AW), 2.0 ** 30, jnp.float32)
                       for _ in range(K))

        def ins_step(u, carry):
            av, ai = carry
            av = list(av)
            ai = list(ai)
            x = cost_scr[u, qb * SQ:(qb + 1) * SQ, :]
            gx = lane + (u * AW).astype(jnp.float32)
            m = [x < av[j] for j in range(K)]
            for j in range(K - 1, 0, -1):
                av[j] = jnp.where(m[j - 1], av[j - 1],
                                  jnp.where(m[j], x, av[j]))
                ai[j] = jnp.where(m[j - 1], ai[j - 1],
                                  jnp.where(m[j], gx, ai[j]))
            av[0] = jnp.where(m[0], x, av[0])
            ai[0] = jnp.where(m[0], gx, ai[0])
            return tuple(av), tuple(ai)

        av, ai = jax.lax.fori_loop(0, NT * NSUB, ins_step,
                                   (acc_v0, acc_i0), unroll=32)
        av_ref[qb * SQ:(qb + 1) * SQ, :] = jnp.concatenate(av, axis=1)
        ai_ref[qb * SQ:(qb + 1) * SQ, :] = jnp.concatenate(ai, axis=1)


BQM = 256


def _merge_body(av_ref, ai_ref, wgt_ref, idx_ref):
    fv = av_ref[...]
    fi = ai_ref[...]
    inf = jnp.float32(jnp.inf)
    ms = []
    idxs = []
    m_prev = jnp.full((BQM, 1), -inf, dtype=jnp.float32)
    i_prev = jnp.full((BQM, 1), -1.0, dtype=jnp.float32)
    for k in range(K):
        valid = (fv > m_prev) | ((fv == m_prev) & (fi > i_prev))
        ceff = jnp.where(valid, fv, inf)
        m = jnp.min(ceff, axis=1, keepdims=True)
        i = jnp.min(jnp.where(ceff == m, fi, jnp.float32(2.0 ** 30)),
                    axis=1, keepdims=True)
        ms.append(m)
        idxs.append(i)
        m_prev, i_prev = m, i

    costs = jnp.concatenate(ms, axis=1)
    ids = jnp.concatenate(idxs, axis=1)
    e = jnp.exp(costs[:, 0:1] - costs)
    wgt_ref[...] = e / jnp.sum(e, axis=1, keepdims=True)
    idx_ref[...] = ids.astype(jnp.int32)


def _topk(qf, kt3, n):
    av, ai = pl.pallas_call(
        _topk_body,
        grid=(n // BQ,),
        in_specs=[
            pl.BlockSpec((BQ, CF), lambda i: (i, 0)),
            pl.BlockSpec((NT, CF, CW), lambda i: (0, 0, 0)),
        ],
        out_specs=[
            pl.BlockSpec((BQ, K * AW), lambda i: (i, 0)),
            pl.BlockSpec((BQ, K * AW), lambda i: (i, 0)),
        ],
        out_shape=[
            jax.ShapeDtypeStruct((n, K * AW), jnp.float32),
            jax.ShapeDtypeStruct((n, K * AW), jnp.float32),
        ],
        scratch_shapes=[pltpu.VMEM((N // AW, BQ, AW), jnp.float32)],
    )(qf, kt3)
    return pl.pallas_call(
        _merge_body,
        grid=(n // BQM,),
        in_specs=[
            pl.BlockSpec((BQM, K * AW), lambda i: (i, 0)),
            pl.BlockSpec((BQM, K * AW), lambda i: (i, 0)),
        ],
        out_specs=[
            pl.BlockSpec((BQM, K), lambda i: (i, 0)),
            pl.BlockSpec((BQM, K), lambda i: (i, 0)),
        ],
        out_shape=[
            jax.ShapeDtypeStruct((n, K), jnp.float32),
            jax.ShapeDtypeStruct((n, K), jnp.int32),
        ],
    )(av, ai)


def _sc_att(vT, idx, wgt, n):
    NI = n * K
    QW = 32
    VD = 128
    vpad = jnp.pad(vT, ((0, 0), (0, VD - CF)))
    wB = jnp.broadcast_to(wgt.reshape(n, K, 1), (n, K, CF)).reshape(n, K * CF)
    mesh = plsc.VectorSubcoreMesh(core_axis_name="core",
                                  subcore_axis_name="subcore")

    @pl.kernel(out_type=jax.ShapeDtypeStruct((n // 8, VD), jnp.float32),
               mesh=mesh,
               scratch_types=[pltpu.VMEM((QW * K, VD), jnp.float32)])
    def gk(v_hbm, i_hbm, w_hbm, o_hbm, g_scr):
        def body(i_vmem, w_vmem, o_vmem):
            pltpu.sync_copy(v_hbm.at[i_vmem.at[0]], g_scr)

            @pl.loop(0, QW // 8)
            def _(q0):
                for qq in range(8):
                    q = q0 * 8 + qq
                    acc = w_vmem[q, 0:CF] * g_scr[q * K, 0:CF]
                    for k in range(1, K):
                        acc = acc + (w_vmem[q, k * CF:(k + 1) * CF]
                                     * g_scr[q * K + k, 0:CF])
                    o_vmem[q0, qq * CF:(qq + 1) * CF] = acc

        pltpu.emit_pipeline(
            body,
            grid=(NI // (QW * K),),
            in_specs=[pl.BlockSpec((1, QW * K), index_map=lambda i: (0, i)),
                      pl.BlockSpec((QW, VD), index_map=lambda i: (i, 0))],
            out_specs=[pl.BlockSpec((QW // 8, VD), index_map=lambda i: (i, 0))],
            core_axis_name='subcore',
            dimension_semantics=(pltpu.PARALLEL,),
        )(i_hbm, w_hbm, o_hbm)

    out = gk(vpad, idx.reshape(1, NI), wB)
    return out.reshape(n, CF)


def kernel(a, b, Wq, bq, Wk, bk, Wv, bv, Wf, bf):
    q = jax.nn.relu(_conv(a, Wq, bq))[0]
    k = jax.nn.relu(_conv(b, Wk, bk))[0]
    v = jax.nn.relu(_conv(b, Wv, bv))[0]

    qf = q.reshape(CF, N).T
    kt3 = k.reshape(CF, NT, CW).transpose(1, 0, 2)
    vT = v.reshape(CF, N).T

    NS = 4
    n = N // NS
    att_parts = []
    for p in range(NS):
        wgt, idx = _topk(qf[p * n:(p + 1) * n], kt3, n)
        att_parts.append(_sc_att(vT, idx, wgt, n))
    att = jnp.concatenate(att_parts, axis=0)
    att = att.T.reshape(1, CF, H, W)

    out = jax.nn.sigmoid(_conv(jnp.concatenate([a, att], axis=1), Wf, bf))
    return out


# speedup vs baseline: 1.8270x; 1.0086x over previous
import jax
import jax.numpy as jnp
from jax.experimental import pallas as pl
from jax.experimental.pallas import tpu as pltpu
from jax.experimental.pallas import tpu_sc as plsc

H = 128
W = 128
CF = 16
K = 8
N = H * W
BQ = 128
SQ = 16
CW = 1024
NT = N // CW


def _conv(x, w, b):
    y = jax.lax.conv_general_dilated(
        x, w, (1, 1), 'SAME', dimension_numbers=('NCHW', 'OIHW', 'NCHW'))
    return y + b[None, :, None, None]


AW = 128
NSUB = CW // AW


def _topk_body(q_ref, kt_ref, av_ref, ai_ref, cost_scr):
    q = q_ref[...]
    q2 = jnp.sum(q * q, axis=1, keepdims=True)

    inf = jnp.float32(jnp.inf)
    lane = jax.lax.broadcasted_iota(
        jnp.int32, (SQ, AW), 1).astype(jnp.float32)

    def mm_step(t, carry):
        for s in range(NSUB):
            kts = kt_ref[t, :, s * AW:(s + 1) * AW]
            k2 = jnp.sum(kts * kts, axis=0, keepdims=True)
            cost_scr[t * NSUB + s] = (
                q2 - 2.0 * jnp.dot(q, kts,
                                   preferred_element_type=jnp.float32) + k2)
        return carry

    jax.lax.fori_loop(0, NT, mm_step, 0, unroll=8)

    for qb in range(BQ // SQ):
        acc_v0 = tuple(jnp.full((SQ, AW), inf, jnp.float32)
                       for _ in range(K))
        acc_i0 = tuple(jnp.full((SQ, AW), 2.0 ** 30, jnp.float32)
                       for _ in range(K))

        def ins_step(u, carry):
            av, ai = carry
            av = list(av)
            ai = list(ai)
            x = cost_scr[u, qb * SQ:(qb + 1) * SQ, :]
            gx = lane + (u * AW).astype(jnp.float32)
            m = [x < av[j] for j in range(K)]
            for j in range(K - 1, 0, -1):
                av[j] = jnp.where(m[j - 1], av[j - 1],
                                  jnp.where(m[j], x, av[j]))
                ai[j] = jnp.where(m[j - 1], ai[j - 1],
                                  jnp.where(m[j], gx, ai[j]))
            av[0] = jnp.where(m[0], x, av[0])
            ai[0] = jnp.where(m[0], gx, ai[0])
            return tuple(av), tuple(ai)

        av, ai = jax.lax.fori_loop(0, NT * NSUB, ins_step,
                                   (acc_v0, acc_i0), unroll=64)
        av_ref[qb * SQ:(qb + 1) * SQ, :] = jnp.concatenate(av, axis=1)
        ai_ref[qb * SQ:(qb + 1) * SQ, :] = jnp.concatenate(ai, axis=1)


BQM = 256


def _merge_body(av_ref, ai_ref, wgt_ref, idx_ref):
    fv = av_ref[...]
    fi = ai_ref[...]
    inf = jnp.float32(jnp.inf)
    ms = []
    idxs = []
    m_prev = jnp.full((BQM, 1), -inf, dtype=jnp.float32)
    i_prev = jnp.full((BQM, 1), -1.0, dtype=jnp.float32)
    for k in range(K):
        valid = (fv > m_prev) | ((fv == m_prev) & (fi > i_prev))
        ceff = jnp.where(valid, fv, inf)
        m = jnp.min(ceff, axis=1, keepdims=True)
        i = jnp.min(jnp.where(ceff == m, fi, jnp.float32(2.0 ** 30)),
                    axis=1, keepdims=True)
        ms.append(m)
        idxs.append(i)
        m_prev, i_prev = m, i

    costs = jnp.concatenate(ms, axis=1)
    ids = jnp.concatenate(idxs, axis=1)
    e = jnp.exp(costs[:, 0:1] - costs)
    wgt_ref[...] = e / jnp.sum(e, axis=1, keepdims=True)
    idx_ref[...] = ids.astype(jnp.int32)


def _topk(qf, kt3, n):
    av, ai = pl.pallas_call(
        _topk_body,
        grid=(n // BQ,),
        in_specs=[
            pl.BlockSpec((BQ, CF), lambda i: (i, 0)),
            pl.BlockSpec((NT, CF, CW), lambda i: (0, 0, 0)),
        ],
        out_specs=[
            pl.BlockSpec((BQ, K * AW), lambda i: (i, 0)),
            pl.BlockSpec((BQ, K * AW), lambda i: (i, 0)),
        ],
        out_shape=[
            jax.ShapeDtypeStruct((n, K * AW), jnp.float32),
            jax.ShapeDtypeStruct((n, K * AW), jnp.float32),
        ],
        scratch_shapes=[pltpu.VMEM((N // AW, BQ, AW), jnp.float32)],
    )(qf, kt3)
    return pl.pallas_call(
        _merge_body,
        grid=(n // BQM,),
        in_specs=[
            pl.BlockSpec((BQM, K * AW), lambda i: (i, 0)),
            pl.BlockSpec((BQM, K * AW), lambda i: (i, 0)),
        ],
        out_specs=[
            pl.BlockSpec((BQM, K), lambda i: (i, 0)),
            pl.BlockSpec((BQM, K), lambda i: (i, 0)),
        ],
        out_shape=[
            jax.ShapeDtypeStruct((n, K), jnp.float32),
            jax.ShapeDtypeStruct((n, K), jnp.int32),
        ],
    )(av, ai)


def _sc_att(vT, idx, wgt, n):
    NI = n * K
    QW = 32
    VD = 128
    vpad = jnp.pad(vT, ((0, 0), (0, VD - CF)))
    wB = jnp.broadcast_to(wgt.reshape(n, K, 1), (n, K, CF)).reshape(n, K * CF)
    mesh = plsc.VectorSubcoreMesh(core_axis_name="core",
                                  subcore_axis_name="subcore")

    @pl.kernel(out_type=jax.ShapeDtypeStruct((n // 8, VD), jnp.float32),
               mesh=mesh,
               scratch_types=[pltpu.VMEM((QW * K, VD), jnp.float32)])
    def gk(v_hbm, i_hbm, w_hbm, o_hbm, g_scr):
        def body(i_vmem, w_vmem, o_vmem):
            pltpu.sync_copy(v_hbm.at[i_vmem.at[0]], g_scr)

            @pl.loop(0, QW // 8)
            def _(q0):
                for qq in range(8):
                    q = q0 * 8 + qq
                    acc = w_vmem[q, 0:CF] * g_scr[q * K, 0:CF]
                    for k in range(1, K):
                        acc = acc + (w_vmem[q, k * CF:(k + 1) * CF]
                                     * g_scr[q * K + k, 0:CF])
                    o_vmem[q0, qq * CF:(qq + 1) * CF] = acc

        pltpu.emit_pipeline(
            body,
            grid=(NI // (QW * K),),
            in_specs=[pl.BlockSpec((1, QW * K), index_map=lambda i: (0, i)),
                      pl.BlockSpec((QW, VD), index_map=lambda i: (i, 0))],
            out_specs=[pl.BlockSpec((QW // 8, VD), index_map=lambda i: (i, 0))],
            core_axis_name='subcore',
            dimension_semantics=(pltpu.PARALLEL,),
        )(i_hbm, w_hbm, o_hbm)

    out = gk(vpad, idx.reshape(1, NI), wB)
    return out.reshape(n, CF)


def kernel(a, b, Wq, bq, Wk, bk, Wv, bv, Wf, bf):
    q = jax.nn.relu(_conv(a, Wq, bq))[0]
    k = jax.nn.relu(_conv(b, Wk, bk))[0]
    v = jax.nn.relu(_conv(b, Wv, bv))[0]

    qf = q.reshape(CF, N).T
    kt3 = k.reshape(CF, NT, CW).transpose(1, 0, 2)
    vT = v.reshape(CF, N).T

    NS = 4
    n = N // NS
    att_parts = []
    for p in range(NS):
        wgt, idx = _topk(qf[p * n:(p + 1) * n], kt3, n)
        att_parts.append(_sc_att(vT, idx, wgt, n))
    att = jnp.concatenate(att_parts, axis=0)
    att = att.T.reshape(1, CF, H, W)

    out = jax.nn.sigmoid(_conv(jnp.concatenate([a, att], axis=1), Wf, bf))
    return out
